# Initial kernel scaffold; baseline (speedup 1.0000x reference)
import functools

import jax
import jax.numpy as jnp
from jax import lax
from jax.experimental import pallas as pl
from jax.experimental.pallas import tpu as pltpu
from jax.experimental.pallas import tpu_sc as plsc

NC = 2
NS = 16
NW = NC * NS
L = 128
NBUF = 5


def kernel(X, W):
    B, H = X.shape
    V, D = W.shape
    N = B * H
    assert N % (L * NW) == 0
    rows = N // L
    rpw = rows // NW
    assert rpw % NBUF == 0
    ngrp = rpw // NBUF

    Xf = X.reshape(rows, L).astype(jnp.int32)

    mesh = plsc.VectorSubcoreMesh(core_axis_name="c", subcore_axis_name="s")

    @functools.partial(
        pl.kernel,
        out_type=jax.ShapeDtypeStruct((N, D), jnp.float32),
        mesh=mesh,
        scratch_types=[
            pltpu.VMEM((rpw, L), jnp.int32),
            [pltpu.VMEM((L, D), jnp.float32) for _ in range(NBUF)],
            [pltpu.SemaphoreType.DMA for _ in range(NBUF)],
        ],
    )
    def emb(x_hbm, w_hbm, out_hbm, idx_v, bufs, sems):
        wid = lax.axis_index("s") * NC + lax.axis_index("c")
        row0 = wid * rpw
        pltpu.sync_copy(x_hbm.at[pl.ds(row0, rpw)], idx_v)

        def start(b, chunk):
            pltpu.make_async_copy(
                w_hbm.at[idx_v.at[chunk]], bufs[b], sems[b]
            ).start()

        def finish(b, chunk):
            pltpu.make_async_copy(
                w_hbm.at[idx_v.at[chunk]], bufs[b], sems[b]
            ).wait()
            pltpu.sync_copy(bufs[b], out_hbm.at[pl.ds((row0 + chunk) * L, L)])

        for b in range(NBUF):
            start(b, b)

        def grp(g, carry):
            c0 = g * NBUF
            for b in range(NBUF):
                finish(b, c0 + b)
                start(b, c0 + b + NBUF)
            return carry

        lax.fori_loop(0, ngrp - 1, grp, 0)

        c0 = (ngrp - 1) * NBUF
        for b in range(NBUF):
            finish(b, c0 + b)

    out = emb(Xf, W)
    return out.reshape(B, H, D)


# speedup vs baseline: 3.3386x; 3.3386x over previous
import functools

import jax
import jax.numpy as jnp
from jax import lax
from jax.experimental import pallas as pl
from jax.experimental.pallas import tpu as pltpu
from jax.experimental.pallas import tpu_sc as plsc

NC = 2
NS = 16
NW = NC * NS
L = 128
NBUF = 5


def kernel(X, W):
    B, H = X.shape
    V, D = W.shape
    N = B * H
    assert N % (L * NW) == 0
    rows = N // L
    rpw = rows // NW
    assert rpw % NBUF == 0
    ngrp = rpw // NBUF

    Xf = X.reshape(NW, rpw, L).astype(jnp.int32)

    mesh = plsc.VectorSubcoreMesh(core_axis_name="c", subcore_axis_name="s")

    @functools.partial(
        pl.kernel,
        out_type=jax.ShapeDtypeStruct((N, D), jnp.float32),
        mesh=mesh,
        scratch_types=[
            pltpu.VMEM((rpw, L), jnp.int32),
            [pltpu.VMEM((L, D), jnp.float32) for _ in range(NBUF)],
            [pltpu.SemaphoreType.DMA for _ in range(NBUF)],
        ],
    )
    def emb(x_hbm, w_hbm, out_hbm, idx_v, bufs, sems):
        wid = lax.axis_index("s") * NC + lax.axis_index("c")
        row0 = wid * rpw
        pltpu.sync_copy(x_hbm.at[wid], idx_v)

        def start(b, chunk):
            pltpu.make_async_copy(
                w_hbm.at[idx_v.at[chunk]], bufs[b], sems[b]
            ).start()

        def finish(b, chunk):
            pltpu.make_async_copy(
                w_hbm.at[idx_v.at[chunk]], bufs[b], sems[b]
            ).wait()
            pltpu.sync_copy(bufs[b], out_hbm.at[pl.ds((row0 + chunk) * L, L)])

        for b in range(NBUF):
            start(b, b)

        def grp(g, carry):
            c0 = g * NBUF
            for b in range(NBUF):
                finish(b, c0 + b)
                start(b, c0 + b + NBUF)
            return carry

        lax.fori_loop(0, ngrp - 1, grp, 0)

        c0 = (ngrp - 1) * NBUF
        for b in range(NBUF):
            finish(b, c0 + b)

    out = emb(Xf, W)
    return out.reshape(B, H, D)
